# baseline (speedup 1.0000x reference)
import jax
import jax.numpy as jnp
from jax import lax
from jax.experimental import pallas as pl
from jax.experimental.pallas import tpu as pltpu, tpu_sc as plsc

N = 10000
E = 320000
D = 128
H = 64
CHUNK = 128
NTILES = 16
NCH = 2560
EPAD = NCH * CHUNK
JMAX = NCH // NTILES
RPT = N // NTILES
ACC_ROWS = N + 48
NPAD = 10240
CPT = NPAD // NTILES
CW = 16
ZR = 125
ZC = 128
NBUF = 4
NIDX = 8


def _fill2d(ref, rows, width, value):
    w16 = width // 16

    def body(i, _):
        r = i // w16
        k = i % w16
        ref[r, pl.ds(k * 16, 16)] = jnp.full((16,), value, jnp.float32)
        return 0

    lax.fori_loop(0, rows * w16, body, 0)


def _maybe_when(cond, fn):
    if isinstance(cond, bool):
        if cond:
            fn()
    else:
        pl.when(cond)(fn)


def _make_agg(with_count):
    mesh = plsc.VectorSubcoreMesh(core_axis_name="c", subcore_axis_name="s")
    out_type = [jax.ShapeDtypeStruct((2 * N, H), jnp.float32)]
    if with_count:
        out_type.append(jax.ShapeDtypeStruct((2 * N, CW), jnp.float32))
    scratch = (
        [pltpu.VMEM((CHUNK, H), jnp.float32) for _ in range(NBUF)]
        + [pltpu.VMEM((2, CHUNK), jnp.int32) for _ in range(NIDX)]
        + [pltpu.VMEM_SHARED((ACC_ROWS, H), jnp.float32)]
        + [pltpu.SemaphoreType.DMA for _ in range(NBUF)]
        + [pltpu.SemaphoreType.DMA for _ in range(NBUF)]
        + [pltpu.SemaphoreType.DMA for _ in range(NIDX)]
    )
    if with_count:
        scratch += (
            [pltpu.VMEM((CHUNK, CW), jnp.float32),
             pltpu.VMEM((ZC, CW), jnp.float32),
             pltpu.VMEM_SHARED((NPAD, CW), jnp.float32)]
            + [pltpu.SemaphoreType.DMA for _ in range(NBUF)]
        )

    def body(x0_hbm, x1_hbm, eidx_hbm, *refs):
        out_hbm = refs[0]
        k = 2 if with_count else 1
        cnt_hbm = refs[1] if with_count else None
        rows = refs[k:k + NBUF]
        idx2 = refs[k + NBUF:k + NBUF + NIDX]
        base = k + NBUF + NIDX
        acc_sh = refs[base]
        gat_sem = refs[base + 1:base + 1 + NBUF]
        sct_sem = refs[base + 1 + NBUF:base + 1 + 2 * NBUF]
        idx_sem = refs[base + 1 + 2 * NBUF:base + 1 + 2 * NBUF + NIDX]
        if with_count:
            cbase = base + 1 + 2 * NBUF + NIDX
            ones_v, zero_c, cnt_sh = refs[cbase:cbase + 3]
            cnt_sem = refs[cbase + 3:cbase + 3 + NBUF]

        c = lax.axis_index("c")
        s = lax.axis_index("s")
        r0 = s * RPT

        _fill2d(rows[0], CHUNK, H, 0.0)

        def zbody(i, _):
            pltpu.sync_copy(rows[0].at[pl.ds(0, ZR)],
                            acc_sh.at[pl.ds(r0 + i * ZR, ZR)])
            return 0

        lax.fori_loop(0, RPT // ZR, zbody, 0)

        if with_count:
            _fill2d(ones_v, CHUNK, CW, 1.0)
            _fill2d(zero_c, ZC, CW, 0.0)

            def zcbody(i, _):
                pltpu.sync_copy(zero_c,
                                cnt_sh.at[pl.ds(s * CPT + i * ZC, ZC)])
                return 0

            lax.fori_loop(0, CPT // ZC, zcbody, 0)

        plsc.subcore_barrier()

        def chunk_of(jv):
            return s + jv * NTILES

        def fire_idx(jv, v):
            pltpu.async_copy(eidx_hbm.at[chunk_of(jv)], idx2[v], idx_sem[v])

        def wait_idx(jv, v):
            pltpu.make_async_copy(eidx_hbm.at[chunk_of(jv)], idx2[v],
                                  idx_sem[v]).wait()

        def fire_gather(u, v):
            @pl.when(c == 0)
            def _():
                pltpu.async_copy(x0_hbm.at[idx2[v].at[0]], rows[u],
                                 gat_sem[u])

            @pl.when(c == 1)
            def _():
                pltpu.async_copy(x1_hbm.at[idx2[v].at[0]], rows[u],
                                 gat_sem[u])

        def wait_gather(u, v):
            pltpu.make_async_copy(x0_hbm.at[idx2[v].at[0]], rows[u],
                                  gat_sem[u]).wait()

        def fire_scatter(u, v, par):
            pltpu.async_copy(rows[u], acc_sh.at[idx2[v].at[1]], sct_sem[u],
                             add=True)
            if with_count:
                @pl.when(c == par)
                def _():
                    pltpu.async_copy(ones_v, cnt_sh.at[idx2[v].at[1]],
                                     cnt_sem[u], add=True)

        def wait_scatter(u, v, par):
            pltpu.make_async_copy(rows[u], acc_sh.at[idx2[v].at[1]],
                                  sct_sem[u]).wait()
            if with_count:
                @pl.when(c == par)
                def _():
                    pltpu.make_async_copy(ones_v, cnt_sh.at[idx2[v].at[1]],
                                          cnt_sem[u]).wait()

        def do_slot(jv, i, has_prev):
            u = i % NBUF
            v = i % NIDX
            u2 = (i + 2) % NBUF
            v2 = (i + 2) % NIDX
            v6 = (i + 6) % NIDX
            wait_gather(u, v)
            fire_scatter(u, v, i % 2)

            def prep_next():
                if has_prev:
                    wait_scatter(u2, v6, i % 2)

                def pf():
                    fire_idx(jv + 6, v6)

                _maybe_when(jv + 6 < JMAX, pf)
                wait_idx(jv + 2, v2)
                fire_gather(u2, v2)

            _maybe_when(jv + 2 < JMAX, prep_next)

        for j in range(6):
            fire_idx(j, j)
        wait_idx(0, 0)
        fire_gather(0, 0)
        wait_idx(1, 1)
        fire_gather(1, 1)
        for j in range(8):
            do_slot(j, j, j >= 2)

        def mbody(kk, _):
            for i in range(8):
                do_slot(kk * 8 + i, i, True)
            return 0

        lax.fori_loop(1, JMAX // 8, mbody, 0)

        for (u, v) in ((0, 4), (1, 5), (2, 6), (3, 7)):
            wait_scatter(u, v, v % 2)

        plsc.subcore_barrier()

        pltpu.sync_copy(acc_sh.at[pl.ds(r0, RPT)],
                        out_hbm.at[pl.ds(c * N + r0, RPT)])
        if with_count:
            @pl.when(s < NTILES - 1)
            def _():
                pltpu.sync_copy(cnt_sh.at[pl.ds(s * CPT, CPT)],
                                cnt_hbm.at[pl.ds(c * N + s * CPT, CPT)])

            @pl.when(s == NTILES - 1)
            def _():
                pltpu.sync_copy(cnt_sh.at[pl.ds(s * CPT, N - s * CPT)],
                                cnt_hbm.at[pl.ds(c * N + s * CPT,
                                                 N - s * CPT)])

    return pl.kernel(body, out_type=out_type, mesh=mesh,
                     scratch_types=scratch,
                     compiler_params=pltpu.CompilerParams(
                         use_tc_tiling_on_sc=False))


_agg_with_count = _make_agg(True)
_agg_no_count = _make_agg(False)


def _make_layer(residual, split_out):
    BLK = 1000

    def body(x_ref, n0_ref, n1_ref, c0_ref, c1_ref, wa_ref, wb_ref, b_ref,
             *o_refs):
        cval = c0_ref[...][:, 0:1] + c1_ref[...][:, 0:1]
        cval = jnp.where(cval == 0.0, 1.0, cval)
        nm = jnp.concatenate([n0_ref[...], n1_ref[...]], axis=-1) / cval
        y = (jnp.dot(x_ref[...], wa_ref[...],
                     preferred_element_type=jnp.float32)
             + jnp.dot(nm, wb_ref[...], preferred_element_type=jnp.float32)
             + b_ref[...])
        y = jnp.maximum(y, 0.0)
        if residual:
            y = y + x_ref[...]
        o_refs[0][...] = y
        if split_out:
            o_refs[1][...] = y[:, :H]
            o_refs[2][...] = y[:, H:]

    nb = N // BLK
    out_shape = [jax.ShapeDtypeStruct((N, D), jnp.float32)]
    out_specs = [pl.BlockSpec((BLK, D), lambda i: (i, 0))]
    if split_out:
        out_shape += [jax.ShapeDtypeStruct((N, H), jnp.float32)] * 2
        out_specs += [pl.BlockSpec((BLK, H), lambda i: (i, 0))] * 2
    return pl.pallas_call(
        body,
        grid=(nb,),
        in_specs=[
            pl.BlockSpec((BLK, D), lambda i: (i, 0)),
            pl.BlockSpec((BLK, H), lambda i: (i, 0)),
            pl.BlockSpec((BLK, H), lambda i: (i + nb, 0)),
            pl.BlockSpec((BLK, CW), lambda i: (i, 0)),
            pl.BlockSpec((BLK, CW), lambda i: (i + nb, 0)),
            pl.BlockSpec((D, D), lambda i: (0, 0)),
            pl.BlockSpec((D, D), lambda i: (0, 0)),
            pl.BlockSpec((1, D), lambda i: (0, 0)),
        ],
        out_specs=out_specs,
        out_shape=out_shape,
    )


_layer_res = _make_layer(True, True)
_layer_last = _make_layer(False, False)


def kernel(x, edge_index, W1, b1, W2, b2):
    row = edge_index[0]
    col = edge_index[1]
    pad = EPAD - E
    rp = jnp.concatenate([row, jnp.zeros((pad,), jnp.int32)])
    cp = jnp.concatenate([col, jnp.full((pad,), N, jnp.int32)])
    eidx = jnp.stack([rp.reshape(NCH, CHUNK), cp.reshape(NCH, CHUNK)],
                     axis=1)
    w1a = W1[:, :D].T
    w1b = W1[:, D:].T
    w2a = W2[:, :D].T
    w2b = W2[:, D:].T

    agg1, cnt = _agg_with_count(x[:, :H], x[:, H:], eidx)
    h1, h1a, h1b = _layer_res(x, agg1, agg1, cnt, cnt, w1a, w1b,
                              b1.reshape(1, D))
    (agg2,) = _agg_no_count(h1a, h1b, eidx)
    h2 = _layer_last(h1, agg2, agg2, cnt, cnt, w2a, w2b, b2.reshape(1, D))
    return h2


# speedup vs baseline: 1.7977x; 1.7977x over previous
import jax
import jax.numpy as jnp
from jax import lax
from jax.experimental import pallas as pl
from jax.experimental.pallas import tpu as pltpu, tpu_sc as plsc

N = 10000
E = 320000
D = 128
H = 64
CHUNK = 128
NTILES = 16
NCH = 2560
EPAD = NCH * CHUNK
JMAX = NCH // NTILES
RPT = N // NTILES
ACC_ROWS = N + 48
NPAD = 10240
CPT = NPAD // NTILES
CW = 8
ZR = 125
ZC = 128
NBUF = 4
NIDX = 8


def _fill2d(ref, rows, width, value):
    w16 = width // 16

    def body(i, _):
        r = i // w16
        k = i % w16
        ref[r, pl.ds(k * 16, 16)] = jnp.full((16,), value, jnp.float32)
        return 0

    lax.fori_loop(0, rows * w16, body, 0)


def _maybe_when(cond, fn):
    if isinstance(cond, bool):
        if cond:
            fn()
    else:
        pl.when(cond)(fn)


def _make_agg(with_count):
    mesh = plsc.VectorSubcoreMesh(core_axis_name="c", subcore_axis_name="s")
    out_type = [jax.ShapeDtypeStruct((2 * N, H), jnp.float32)]
    if with_count:
        out_type.append(jax.ShapeDtypeStruct((2 * N, CW), jnp.float32))
    scratch = (
        [pltpu.VMEM((CHUNK, H), jnp.float32) for _ in range(NBUF)]
        + [pltpu.VMEM((2, CHUNK), jnp.int32) for _ in range(NIDX)]
        + [pltpu.VMEM_SHARED((N, H), jnp.float32),
           pltpu.VMEM_SHARED((ACC_ROWS, H), jnp.float32)]
        + [pltpu.SemaphoreType.DMA for _ in range(NBUF)]
        + [pltpu.SemaphoreType.DMA for _ in range(NBUF)]
        + [pltpu.SemaphoreType.DMA for _ in range(NIDX)]
    )
    if with_count:
        scratch += (
            [pltpu.VMEM((CHUNK, CW), jnp.float32),
             pltpu.VMEM_SHARED((NPAD, CW), jnp.float32)]
            + [pltpu.SemaphoreType.DMA for _ in range(NBUF)]
        )

    def body(x_hbm, eidx_hbm, *refs):
        if with_count:
            ones_hbm, zeros_hbm = refs[0], refs[1]
            refs = refs[2:]
        out_hbm = refs[0]
        k = 2 if with_count else 1
        cnt_hbm = refs[1] if with_count else None
        rows = refs[k:k + NBUF]
        idx2 = refs[k + NBUF:k + NBUF + NIDX]
        base = k + NBUF + NIDX
        x_sh, acc_sh = refs[base], refs[base + 1]
        gat_sem = refs[base + 2:base + 2 + NBUF]
        sct_sem = refs[base + 2 + NBUF:base + 2 + 2 * NBUF]
        idx_sem = refs[base + 2 + 2 * NBUF:base + 2 + 2 * NBUF + NIDX]
        if with_count:
            cbase = base + 2 + 2 * NBUF + NIDX
            ones_v, cnt_sh = refs[cbase:cbase + 2]
            cnt_sem = refs[cbase + 2:cbase + 2 + NBUF]

        c = lax.axis_index("c")
        s = lax.axis_index("s")
        r0 = s * RPT

        _fill2d(rows[0], CHUNK, H, 0.0)

        def zbody(i, _):
            pltpu.sync_copy(rows[0].at[pl.ds(0, ZR)],
                            acc_sh.at[pl.ds(r0 + i * ZR, ZR)])
            return 0

        lax.fori_loop(0, RPT // ZR, zbody, 0)
        pltpu.sync_copy(x_hbm.at[pl.ds(r0, RPT), pl.ds(c * H, H)],
                        x_sh.at[pl.ds(r0, RPT)])

        if with_count:
            pltpu.sync_copy(ones_hbm, ones_v)

            def zcbody(i, _):
                pltpu.sync_copy(zeros_hbm,
                                cnt_sh.at[pl.ds(s * CPT + i * ZC, ZC)])
                return 0

            lax.fori_loop(0, CPT // ZC, zcbody, 0)

        plsc.subcore_barrier()

        def chunk_of(jv):
            return s + jv * NTILES

        def fire_idx(jv, v):
            pltpu.async_copy(eidx_hbm.at[chunk_of(jv)], idx2[v], idx_sem[v])

        def wait_idx(jv, v):
            pltpu.make_async_copy(eidx_hbm.at[chunk_of(jv)], idx2[v],
                                  idx_sem[v]).wait()

        def fire_gather(u, v):
            pltpu.async_copy(x_sh.at[idx2[v].at[0]], rows[u], gat_sem[u])

        def wait_gather(u, v):
            pltpu.make_async_copy(x_sh.at[idx2[v].at[0]], rows[u],
                                  gat_sem[u]).wait()

        def fire_scatter(u, v, par):
            pltpu.async_copy(rows[u], acc_sh.at[idx2[v].at[1]], sct_sem[u],
                             add=True)
            if with_count:
                @pl.when(c == par)
                def _():
                    pltpu.async_copy(ones_v, cnt_sh.at[idx2[v].at[1]],
                                     cnt_sem[u], add=True)

        def wait_scatter(u, v, par):
            pltpu.make_async_copy(rows[u], acc_sh.at[idx2[v].at[1]],
                                  sct_sem[u]).wait()
            if with_count:
                @pl.when(c == par)
                def _():
                    pltpu.make_async_copy(ones_v, cnt_sh.at[idx2[v].at[1]],
                                          cnt_sem[u]).wait()

        def do_slot(jv, i, has_prev):
            u = i % NBUF
            v = i % NIDX
            u2 = (i + 2) % NBUF
            v2 = (i + 2) % NIDX
            v6 = (i + 6) % NIDX
            wait_gather(u, v)
            fire_scatter(u, v, i % 2)

            def prep_next():
                if has_prev:
                    wait_scatter(u2, v6, i % 2)

                def pf():
                    fire_idx(jv + 6, v6)

                _maybe_when(jv + 6 < JMAX, pf)
                wait_idx(jv + 2, v2)
                fire_gather(u2, v2)

            _maybe_when(jv + 2 < JMAX, prep_next)

        for j in range(6):
            fire_idx(j, j)
        wait_idx(0, 0)
        fire_gather(0, 0)
        wait_idx(1, 1)
        fire_gather(1, 1)
        for j in range(8):
            do_slot(j, j, j >= 2)

        def mbody(kk, _):
            for i in range(8):
                do_slot(kk * 8 + i, i, True)
            return 0

        lax.fori_loop(1, JMAX // 8, mbody, 0)

        for (u, v) in ((0, 4), (1, 5), (2, 6), (3, 7)):
            wait_scatter(u, v, v % 2)

        plsc.subcore_barrier()

        pltpu.sync_copy(acc_sh.at[pl.ds(r0, RPT)],
                        out_hbm.at[pl.ds(c * N + r0, RPT)])
        if with_count:
            @pl.when(s < NTILES - 1)
            def _():
                pltpu.sync_copy(cnt_sh.at[pl.ds(s * CPT, CPT)],
                                cnt_hbm.at[pl.ds(c * N + s * CPT, CPT)])

            @pl.when(s == NTILES - 1)
            def _():
                pltpu.sync_copy(cnt_sh.at[pl.ds(s * CPT, N - s * CPT)],
                                cnt_hbm.at[pl.ds(c * N + s * CPT,
                                                 N - s * CPT)])

    return pl.kernel(body, out_type=out_type, mesh=mesh,
                     scratch_types=scratch,
                     compiler_params=pltpu.CompilerParams(
                         use_tc_tiling_on_sc=False))


_agg_with_count = _make_agg(True)
_agg_no_count = _make_agg(False)


def _make_layer(residual):
    BLK = 1000

    def body(x_ref, n0_ref, n1_ref, c0_ref, c1_ref, wa_ref, wb_ref, b_ref,
             o_ref):
        cval = c0_ref[...][:, 0:1] + c1_ref[...][:, 0:1]
        cval = jnp.where(cval == 0.0, 1.0, cval)
        nm = jnp.concatenate([n0_ref[...], n1_ref[...]], axis=-1) / cval
        y = (jnp.dot(x_ref[...], wa_ref[...],
                     preferred_element_type=jnp.float32)
             + jnp.dot(nm, wb_ref[...], preferred_element_type=jnp.float32)
             + b_ref[...])
        y = jnp.maximum(y, 0.0)
        if residual:
            y = y + x_ref[...]
        o_ref[...] = y

    nb = N // BLK
    return pl.pallas_call(
        body,
        grid=(nb,),
        in_specs=[
            pl.BlockSpec((BLK, D), lambda i: (i, 0)),
            pl.BlockSpec((BLK, H), lambda i: (i, 0)),
            pl.BlockSpec((BLK, H), lambda i: (i + nb, 0)),
            pl.BlockSpec((BLK, CW), lambda i: (i, 0)),
            pl.BlockSpec((BLK, CW), lambda i: (i + nb, 0)),
            pl.BlockSpec((D, D), lambda i: (0, 0)),
            pl.BlockSpec((D, D), lambda i: (0, 0)),
            pl.BlockSpec((1, D), lambda i: (0, 0)),
        ],
        out_specs=pl.BlockSpec((BLK, D), lambda i: (i, 0)),
        out_shape=jax.ShapeDtypeStruct((N, D), jnp.float32),
    )


_layer_res = _make_layer(True)
_layer_last = _make_layer(False)


def kernel(x, edge_index, W1, b1, W2, b2):
    row = edge_index[0]
    col = edge_index[1]
    pad = EPAD - E
    rp = jnp.concatenate([row, jnp.zeros((pad,), jnp.int32)])
    cp = jnp.concatenate([col, jnp.full((pad,), N, jnp.int32)])
    eidx = jnp.stack([rp.reshape(NCH, CHUNK), cp.reshape(NCH, CHUNK)],
                     axis=1)
    w1a = W1[:, :D].T
    w1b = W1[:, D:].T
    w2a = W2[:, :D].T
    w2b = W2[:, D:].T

    ones_c = jnp.ones((CHUNK, CW), jnp.float32)
    zeros_c = jnp.zeros((ZC, CW), jnp.float32)
    agg1, cnt = _agg_with_count(x, eidx, ones_c, zeros_c)
    h1 = _layer_res(x, agg1, agg1, cnt, cnt, w1a, w1b, b1.reshape(1, D))
    (agg2,) = _agg_no_count(h1, eidx)
    h2 = _layer_last(h1, agg2, agg2, cnt, cnt, w2a, w2b, b2.reshape(1, D))
    return h2
